# baseline (speedup 1.0000x reference)
import functools

import jax
import jax.numpy as jnp
from jax.experimental import pallas as pl
from jax.experimental.pallas import tpu as pltpu

_B, _C, _D, _H, _W = 2, 20, 512, 112, 448
_HW = _H * _W
_BLK = 3584
_NBLK = _HW // _BLK
_TOPK = 25
_CP = 32
_HD = _D // 2


def _build_w(thr_ref, labb_ref, cam_ref, w_ref, vcm_ref):
    hig = thr_ref[0, 0]
    low = thr_ref[0, 1]
    bg = thr_ref[0, 2]
    lab = labb_ref[0]
    cam = cam_ref[0]
    vc = lab * cam
    v0 = jnp.max(vc, axis=0, keepdims=True)
    iota_c = jax.lax.broadcasted_iota(jnp.int32, (_C, _HW), 0)
    i0 = jnp.min(jnp.where(vc == v0, iota_c, _C), axis=0, keepdims=True)
    first = iota_c == i0
    v1 = jnp.max(jnp.where(first, -jnp.inf, vc), axis=0, keepdims=True)
    keep = ((v0 >= hig) & (v0 >= low) & (v0 >= bg)
            & ~(((v0 - v1) < 0.3) & (v0 > hig)))
    onehot = (first & keep).astype(jnp.float32)
    count = jnp.sum(onehot, axis=1, keepdims=True)

    need = jnp.any((lab > 0) & (count == 0))
    trip = jnp.where(need, _TOPK, 0)
    vcm_ref[:, :] = vc
    iota_h = jax.lax.broadcasted_iota(jnp.int32, (_C, _HW), 1)

    def body(_, carry):
        vcm = vcm_ref[:, :]
        m = jnp.max(vcm, axis=1, keepdims=True)
        selid = jnp.min(jnp.where(vcm == m, iota_h, _HW), axis=1,
                        keepdims=True)
        vcm_ref[:, :] = jnp.where(iota_h == selid, -1.0, vcm)
        return carry

    jax.lax.fori_loop(0, trip, body, 0)
    sel25 = (vcm_ref[:, :] < 0).astype(jnp.float32)
    w_ref[:, :] = jnp.where(count > 0, onehot / jnp.maximum(count, 1.0),
                            sel25 * (1.0 / _TOPK))


def _loss(lab_ref, wcls_ref, fc_ref, fs_ref, out_ref):
    eyeb = (jax.lax.broadcasted_iota(jnp.int32, (_C, _C), 0)
            == jax.lax.broadcasted_iota(jnp.int32, (_C, _C), 1))
    eyef = eyeb.astype(jnp.float32)
    fc = fc_ref[:, :]
    wcls = wcls_ref[:, :]
    loss_ccf = jnp.zeros((1, 1), jnp.float32)
    loss_cls = jnp.zeros((1, 1), jnp.float32)
    for i in range(_B):
        act = lab_ref[i] > 0
        actf = act.astype(jnp.float32)
        fs = fs_ref[i * _CP:i * _CP + _C, :] * actf
        fsn = fs / jnp.maximum(
            jnp.sqrt(jnp.sum(fs * fs, axis=1, keepdims=True)), 1e-12)
        fcn = fc / jnp.maximum(
            jnp.sqrt(jnp.sum(fc * fc, axis=1, keepdims=True)), 1e-12)
        cosc = jnp.clip(jnp.abs(jax.lax.dot_general(
            fsn, fcn, (((1,), (1,)), ((), ())),
            precision=jax.lax.Precision.HIGHEST,
            preferred_element_type=jnp.float32)), 1e-5, 1.0 - 1e-5)
        ident = eyef * actf
        cond = jnp.all(jnp.where(eyeb, 0.0, cosc) < 0.6, axis=1,
                       keepdims=True)
        sel = act & cond
        self_f = sel.astype(jnp.float32)
        preds = jax.lax.dot_general(
            fs, wcls, (((1,), (1,)), ((), ())),
            precision=jax.lax.Precision.HIGHEST,
            preferred_element_type=jnp.float32)
        pmax = jnp.max(preds, axis=1, keepdims=True)
        e = jnp.exp(preds - pmax)
        p = e / jnp.sum(e, axis=1, keepdims=True)
        p = jnp.clip(p, 1e-12, 1.0 - 1e-12)
        bce_rows = jnp.mean(
            -(eyef * jnp.log(p) + (1.0 - eyef) * jnp.log(1.0 - p)),
            axis=1, keepdims=True)
        loss_cls = loss_cls + jnp.sum(self_f * bce_rows, keepdims=True)
        loss_ccf = loss_ccf + jnp.mean(
            -(ident * jnp.log(cosc)
              + (1.0 - ident) * jnp.log(1.0 - cosc)), keepdims=True)
        n = jnp.sum(self_f, keepdims=True)
        fc = jnp.where(sel, 0.95 * fc + 0.05 * fs, fc)
        loss_cls = jnp.where(n > 0, loss_cls / jnp.maximum(n, 1.0), loss_cls)
    out_ref[:, :] = loss_ccf + loss_cls


def _fused_kernel(thr_ref, labb_ref, lab_ref, cam_ref, f1_ref, f2_ref,
                  wcls_ref, fc_ref, out_ref, w_ref, vcm_ref, fs_ref):
    b = pl.program_id(0)
    k = pl.program_id(1)

    @pl.when(k == 0)
    def _():
        _build_w(thr_ref, labb_ref, cam_ref, w_ref, vcm_ref)
        fs_ref[pl.ds(b * _CP, _C), :] = jnp.zeros((_C, _D), jnp.float32)

    wb = w_ref[:, pl.ds(k * _BLK, _BLK)]
    acc1 = jax.lax.dot_general(
        wb, f1_ref[0, 0], (((1,), (1,)), ((), ())),
        precision=jax.lax.Precision.DEFAULT,
        preferred_element_type=jnp.float32)
    acc2 = jax.lax.dot_general(
        wb, f2_ref[0, 0], (((1,), (1,)), ((), ())),
        precision=jax.lax.Precision.DEFAULT,
        preferred_element_type=jnp.float32)
    fs_ref[pl.ds(b * _CP, _C), :_HD] += acc1
    fs_ref[pl.ds(b * _CP, _C), _HD:] += acc2

    @pl.when((b == _B - 1) & (k == _NBLK - 1))
    def _():
        _loss(lab_ref, wcls_ref, fc_ref, fs_ref, out_ref)


@functools.partial(jax.jit, static_argnames=("interpret",))
def _run(fmap, cam, cls_label, hig, low, bg, W, feature_contrast,
         interpret=False):
    cam3 = cam.reshape(_B, _C, _HW)
    lab3 = cls_label.astype(jnp.float32)[:, :, None]
    thr = jnp.stack([jnp.float32(hig), jnp.float32(low),
                     jnp.float32(bg)]).reshape(1, 3)
    fmap4 = fmap.reshape(_B, 2, _HD, _HW)

    loss = pl.pallas_call(
        _fused_kernel,
        grid=(_B, _NBLK),
        in_specs=[
            pl.BlockSpec((1, 3), lambda b, k: (0, 0)),
            pl.BlockSpec((1, _C, 1), lambda b, k: (b, 0, 0)),
            pl.BlockSpec((_B, _C, 1), lambda b, k: (0, 0, 0)),
            pl.BlockSpec((1, _C, _HW), lambda b, k: (b, 0, 0)),
            pl.BlockSpec((1, 1, _HD, _BLK), lambda b, k: (b, 0, 0, k)),
            pl.BlockSpec((1, 1, _HD, _BLK), lambda b, k: (b, 1, 0, k)),
            pl.BlockSpec((_C, _D), lambda b, k: (0, 0)),
            pl.BlockSpec((_C, _D), lambda b, k: (0, 0)),
        ],
        out_specs=pl.BlockSpec((1, 1), lambda b, k: (0, 0)),
        out_shape=jax.ShapeDtypeStruct((1, 1), jnp.float32),
        scratch_shapes=[pltpu.VMEM((_C, _HW), jnp.float32),
                        pltpu.VMEM((_C, _HW), jnp.float32),
                        pltpu.VMEM((_B * _CP, _D), jnp.float32)],
        interpret=interpret,
    )(thr, lab3, lab3, cam3, fmap4, fmap4, W, feature_contrast)
    return loss[0, 0]


def kernel(fmap, cam, cls_label, hig_thre, low_thre, bg_thre, W,
           feature_contrast):
    return _run(fmap, cam, cls_label, hig_thre, low_thre, bg_thre, W,
                feature_contrast)


# speedup vs baseline: 2.3631x; 2.3631x over previous
import functools

import jax
import jax.numpy as jnp
from jax.experimental import pallas as pl
from jax.experimental.pallas import tpu as pltpu

_B, _C, _D, _H, _W = 2, 20, 512, 112, 448
_HW = _H * _W
_BLK = 3584
_NBLK = _HW // _BLK
_TOPK = 25
_CP = 32
_HD = _D // 2


def _build_w(thr_ref, labb_ref, cam_ref, w_ref, vcm_ref):
    hig = thr_ref[0, 0]
    low = thr_ref[0, 1]
    bg = thr_ref[0, 2]
    lab = labb_ref[0]
    cam = cam_ref[0]
    vc = lab * cam
    v0 = jnp.max(vc, axis=0, keepdims=True)
    iota_c = jax.lax.broadcasted_iota(jnp.int32, (_C, _HW), 0)
    i0 = jnp.min(jnp.where(vc == v0, iota_c, _C), axis=0, keepdims=True)
    first = iota_c == i0
    v1 = jnp.max(jnp.where(first, -jnp.inf, vc), axis=0, keepdims=True)
    keep = ((v0 >= hig) & (v0 >= low) & (v0 >= bg)
            & ~(((v0 - v1) < 0.3) & (v0 > hig)))
    onehot = (first & keep).astype(jnp.float32)
    count = jnp.sum(onehot, axis=1, keepdims=True)

    need = jnp.any((lab > 0) & (count == 0))
    trip = jnp.where(need, _TOPK, 0)
    vcm_ref[:, :] = vc
    iota_h = jax.lax.broadcasted_iota(jnp.int32, (_C, _HW), 1)

    def body(_, carry):
        vcm = vcm_ref[:, :]
        m = jnp.max(vcm, axis=1, keepdims=True)
        selid = jnp.min(jnp.where(vcm == m, iota_h, _HW), axis=1,
                        keepdims=True)
        vcm_ref[:, :] = jnp.where(iota_h == selid, -1.0, vcm)
        return carry

    jax.lax.fori_loop(0, trip, body, 0)
    sel25 = (vcm_ref[:, :] < 0).astype(jnp.float32)
    w_ref[:, :] = jnp.where(count > 0, onehot / jnp.maximum(count, 1.0),
                            sel25 * (1.0 / _TOPK))


def _loss(lab_ref, wcls_ref, fc_ref, fs_ref, out_ref):
    eyeb = (jax.lax.broadcasted_iota(jnp.int32, (_C, _C), 0)
            == jax.lax.broadcasted_iota(jnp.int32, (_C, _C), 1))
    eyef = eyeb.astype(jnp.float32)
    fc = fc_ref[:, :]
    wcls = wcls_ref[:, :]
    loss_ccf = jnp.zeros((1, 1), jnp.float32)
    loss_cls = jnp.zeros((1, 1), jnp.float32)
    for i in range(_B):
        act = lab_ref[i] > 0
        actf = act.astype(jnp.float32)
        fs = fs_ref[i * _CP:i * _CP + _C, :] * actf
        fsn = fs / jnp.maximum(
            jnp.sqrt(jnp.sum(fs * fs, axis=1, keepdims=True)), 1e-12)
        fcn = fc / jnp.maximum(
            jnp.sqrt(jnp.sum(fc * fc, axis=1, keepdims=True)), 1e-12)
        cosc = jnp.clip(jnp.abs(jax.lax.dot_general(
            fsn, fcn, (((1,), (1,)), ((), ())),
            precision=jax.lax.Precision.HIGHEST,
            preferred_element_type=jnp.float32)), 1e-5, 1.0 - 1e-5)
        ident = eyef * actf
        cond = jnp.all(jnp.where(eyeb, 0.0, cosc) < 0.6, axis=1,
                       keepdims=True)
        sel = act & cond
        self_f = sel.astype(jnp.float32)
        preds = jax.lax.dot_general(
            fs, wcls, (((1,), (1,)), ((), ())),
            precision=jax.lax.Precision.HIGHEST,
            preferred_element_type=jnp.float32)
        pmax = jnp.max(preds, axis=1, keepdims=True)
        e = jnp.exp(preds - pmax)
        p = e / jnp.sum(e, axis=1, keepdims=True)
        p = jnp.clip(p, 1e-12, 1.0 - 1e-12)
        bce_rows = jnp.mean(
            -(eyef * jnp.log(p) + (1.0 - eyef) * jnp.log(1.0 - p)),
            axis=1, keepdims=True)
        loss_cls = loss_cls + jnp.sum(self_f * bce_rows, keepdims=True)
        loss_ccf = loss_ccf + jnp.mean(
            -(ident * jnp.log(cosc)
              + (1.0 - ident) * jnp.log(1.0 - cosc)), keepdims=True)
        n = jnp.sum(self_f, keepdims=True)
        fc = jnp.where(sel, 0.95 * fc + 0.05 * fs, fc)
        loss_cls = jnp.where(n > 0, loss_cls / jnp.maximum(n, 1.0), loss_cls)
    out_ref[:, :] = loss_ccf + loss_cls


def _fused_kernel(thr_ref, labb_ref, lab_ref, cam_ref, f1_ref, f2_ref,
                  wcls_ref, fc_ref, out_ref, w_ref, vcm_ref, fs_ref):
    b = pl.program_id(0)
    k = pl.program_id(1)

    @pl.when(k == 0)
    def _():
        _build_w(thr_ref, labb_ref, cam_ref, w_ref, vcm_ref)
        fs_ref[pl.ds(b * _CP, _C), :] = jnp.zeros((_C, _D), jnp.float32)

    wb = w_ref[:, pl.ds(k * _BLK, _BLK)]
    acc1 = jax.lax.dot_general(
        wb, f1_ref[0], (((1,), (1,)), ((), ())),
        precision=jax.lax.Precision.DEFAULT,
        preferred_element_type=jnp.float32)
    acc2 = jax.lax.dot_general(
        wb, f2_ref[0], (((1,), (1,)), ((), ())),
        precision=jax.lax.Precision.DEFAULT,
        preferred_element_type=jnp.float32)
    fs_ref[pl.ds(b * _CP, _C), :_HD] += acc1
    fs_ref[pl.ds(b * _CP, _C), _HD:] += acc2

    @pl.when((b == _B - 1) & (k == _NBLK - 1))
    def _():
        _loss(lab_ref, wcls_ref, fc_ref, fs_ref, out_ref)


@functools.partial(jax.jit, static_argnames=("interpret",))
def _run(fmap, cam, cls_label, hig, low, bg, W, feature_contrast,
         interpret=False):
    cam3 = cam.reshape(_B, _C, _HW)
    lab3 = cls_label.astype(jnp.float32)[:, :, None]
    thr = jnp.stack([jnp.float32(hig), jnp.float32(low),
                     jnp.float32(bg)]).reshape(1, 3)
    fmap3 = fmap.reshape(_B, _D, _HW)

    loss = pl.pallas_call(
        _fused_kernel,
        grid=(_B, _NBLK),
        in_specs=[
            pl.BlockSpec((1, 3), lambda b, k: (0, 0)),
            pl.BlockSpec((1, _C, 1), lambda b, k: (b, 0, 0)),
            pl.BlockSpec((_B, _C, 1), lambda b, k: (0, 0, 0)),
            pl.BlockSpec((1, _C, _HW), lambda b, k: (b, 0, 0)),
            pl.BlockSpec((1, _HD, _BLK), lambda b, k: (b, 0, k)),
            pl.BlockSpec((1, _HD, _BLK), lambda b, k: (b, 1, k)),
            pl.BlockSpec((_C, _D), lambda b, k: (0, 0)),
            pl.BlockSpec((_C, _D), lambda b, k: (0, 0)),
        ],
        out_specs=pl.BlockSpec((1, 1), lambda b, k: (0, 0)),
        out_shape=jax.ShapeDtypeStruct((1, 1), jnp.float32),
        scratch_shapes=[pltpu.VMEM((_C, _HW), jnp.float32),
                        pltpu.VMEM((_C, _HW), jnp.float32),
                        pltpu.VMEM((_B * _CP, _D), jnp.float32)],
        interpret=interpret,
    )(thr, lab3, lab3, cam3, fmap3, fmap3, W, feature_contrast)
    return loss[0, 0]


def kernel(fmap, cam, cls_label, hig_thre, low_thre, bg_thre, W,
           feature_contrast):
    return _run(fmap, cam, cls_label, hig_thre, low_thre, bg_thre, W,
                feature_contrast)
